# baseline (speedup 1.0000x reference)
import jax
import jax.numpy as jnp
from jax import lax
from jax.experimental import pallas as pl
from jax.experimental.pallas import tpu as pltpu


def _crf_body(stT_ref, mt_ref, mr_ref, loss_ref,
              d_ref, rrev_ref, sk_ref, alogz, agold, alens):
    L = stT_ref.shape[0]
    Bb = stT_ref.shape[2]
    g = pl.program_id(0)
    G = pl.num_programs(0)

    @pl.when(g == 0)
    def _init():
        alogz[0, 0] = 0.0
        agold[0, 0] = 0.0
        alens[0, 0] = 0.0

    x = stT_ref[...]
    gold_part = jnp.sum(jnp.where(mt_ref[...] == 3, x, 0.0))

    lens = jnp.sum(mr_ref[...], axis=0)
    lsum_part = jnp.sum(lens).astype(jnp.float32)
    maxl = jnp.minimum(jnp.max(lens), L - 1)

    isub = lax.broadcasted_iota(jnp.int32, (1, L, 1), 1)
    bit = 1
    while bit < L:
        rolled = jnp.roll(x, -bit, axis=0)
        x = jnp.where((isub & bit) != 0, rolled, x)
        bit *= 2
    sk_ref[...] = x

    neg = jnp.full((1, L, Bb), -jnp.inf, dtype=jnp.float32)
    d_ref[0:1, :, :] = neg
    v1 = sk_ref[1:2, 0:L - 1, :]
    d_ref[1:2, 0:L - 1, :] = v1
    rrev_ref[L - 2:L - 1, 1:L, :] = v1

    for w in range(2, L):
        @pl.when(w <= maxl)
        def _step(w=w):
            n = L - w
            left = d_ref[1:w, 0:n, :]
            right = rrev_ref[L - w:L - 1, w:L, :]
            t = left + right
            mx = jnp.max(t, axis=0)
            lse = mx + jnp.log(jnp.sum(jnp.exp(t - mx[None, :, :]), axis=0))
            val = lse + sk_ref[w, 0:n, :]
            d_ref[w, 0:n, :] = val
            rrev_ref[L - 1 - w, w:L, :] = val

    dcol = d_ref[:, 0, :]
    lensc = jnp.minimum(lens, L - 1)
    wiota = lax.broadcasted_iota(jnp.int32, (L, Bb), 0)
    contrib = jnp.where(wiota == lensc[None, :], dcol, 0.0)
    logz_part = jnp.sum(contrib)

    alogz[0, 0] = alogz[0, 0] + logz_part
    agold[0, 0] = agold[0, 0] + gold_part
    alens[0, 0] = alens[0, 0] + lsum_part

    @pl.when(g == G - 1)
    def _fin():
        acc = jnp.stack([alogz[0, 0], agold[0, 0], alens[0, 0]])
        loss_ref[...] = acc.reshape(1, 3)


def _crf_pallas(scores, mask, target, interpret=False):
    B, L = scores.shape[0], scores.shape[1]
    Bb = min(128, B)
    C = 2 if B % (2 * Bb) == 0 else 1
    Bc = B // C

    accs = []
    for c in range(C):
        sl = slice(c * Bc, (c + 1) * Bc)
        stT = jnp.transpose(scores[sl], (2, 1, 0))
        mt8 = (mask[sl].astype(jnp.uint8)
               | (target[sl].astype(jnp.uint8) << 1))
        mt8T = jnp.transpose(mt8, (2, 1, 0))
        mr0t = jnp.transpose(mask[sl, 0, :].astype(jnp.int32), (1, 0))

        acc = pl.pallas_call(
            _crf_body,
            grid=(Bc // Bb,),
            in_specs=[
                pl.BlockSpec((L, L, Bb), lambda g: (0, 0, g)),
                pl.BlockSpec((L, L, Bb), lambda g: (0, 0, g)),
                pl.BlockSpec((L, Bb), lambda g: (0, g)),
            ],
            out_specs=pl.BlockSpec((1, 3), lambda g: (0, 0)),
            out_shape=jax.ShapeDtypeStruct((1, 3), jnp.float32),
            scratch_shapes=[
                pltpu.VMEM((L, L, Bb), jnp.float32),
                pltpu.VMEM((L, L, Bb), jnp.float32),
                pltpu.VMEM((L, L, Bb), jnp.float32),
                pltpu.SMEM((1, 1), jnp.float32),
                pltpu.SMEM((1, 1), jnp.float32),
                pltpu.SMEM((1, 1), jnp.float32),
            ],
            interpret=interpret,
        )(stT, mt8T, mr0t)
        accs.append(acc)

    tot = accs[0] if C == 1 else accs[0] + accs[1]
    loss = (tot[0, 0] - tot[0, 1]) / tot[0, 2]
    return loss, scores


def kernel(scores, mask, target):
    return _crf_pallas(scores, mask, target)


# speedup vs baseline: 1.0996x; 1.0996x over previous
import jax
import jax.numpy as jnp
from jax import lax
from jax.experimental import pallas as pl
from jax.experimental.pallas import tpu as pltpu


def _crf_body(stT_ref, mt_ref, mr_ref, loss_ref,
              d_ref, rrev_ref, sk_ref, alogz, agold, alens):
    L = stT_ref.shape[0]
    Bb = stT_ref.shape[2]
    g = pl.program_id(0)
    G = pl.num_programs(0)

    @pl.when(g == 0)
    def _init():
        alogz[0, 0] = 0.0
        agold[0, 0] = 0.0
        alens[0, 0] = 0.0

    x = stT_ref[...]
    gold_part = jnp.sum(jnp.where(mt_ref[...] == 3, x, 0.0))

    lens = jnp.sum(mr_ref[...], axis=0)
    lsum_part = jnp.sum(lens).astype(jnp.float32)
    maxl = jnp.minimum(jnp.max(lens), L - 1)

    isub = lax.broadcasted_iota(jnp.int32, (1, L, 1), 1)
    bit = 1
    while bit < L:
        rolled = jnp.roll(x, -bit, axis=0)
        x = jnp.where((isub & bit) != 0, rolled, x)
        bit *= 2
    sk_ref[...] = x

    neg = jnp.full((1, L, Bb), -jnp.inf, dtype=jnp.float32)
    d_ref[0:1, :, :] = neg
    v1 = sk_ref[1:2, 0:L - 1, :]
    d_ref[1:2, 0:L - 1, :] = v1
    rrev_ref[L - 2:L - 1, 1:L, :] = v1

    for w in range(2, L):
        @pl.when(w <= maxl)
        def _step(w=w):
            n = L - w
            left = d_ref[1:w, 0:n, :]
            right = rrev_ref[L - w:L - 1, w:L, :]
            t = left + right
            mx = jnp.max(t, axis=0)
            lse = mx + jnp.log(jnp.sum(jnp.exp(t - mx[None, :, :]), axis=0))
            val = lse + sk_ref[w, 0:n, :]
            d_ref[w, 0:n, :] = val
            rrev_ref[L - 1 - w, w:L, :] = val

    dcol = d_ref[:, 0, :]
    lensc = jnp.minimum(lens, L - 1)
    wiota = lax.broadcasted_iota(jnp.int32, (L, Bb), 0)
    contrib = jnp.where(wiota == lensc[None, :], dcol, 0.0)
    logz_part = jnp.sum(contrib)

    alogz[0, 0] = alogz[0, 0] + logz_part
    agold[0, 0] = agold[0, 0] + gold_part
    alens[0, 0] = alens[0, 0] + lsum_part

    @pl.when(g == G - 1)
    def _fin():
        acc = jnp.stack([alogz[0, 0], agold[0, 0], alens[0, 0]])
        loss_ref[...] = acc.reshape(1, 3)


def _crf_pallas(scores, mask, target, interpret=False):
    B, L = scores.shape[0], scores.shape[1]
    Bb = min(128, B)
    C = 1
    Bc = B // C

    one = lax.optimization_barrier(jnp.float32(1.0))
    accs = []
    for c in range(C):
        sl = slice(c * Bc, (c + 1) * Bc)
        stT = jnp.transpose(scores[sl], (2, 1, 0)) * one
        mt8 = (mask[sl].astype(jnp.uint8)
               | (target[sl].astype(jnp.uint8) << 1))
        mt8T = jnp.transpose(mt8, (2, 1, 0))
        mr0t = jnp.transpose(mask[sl, 0, :].astype(jnp.int32), (1, 0))

        acc = pl.pallas_call(
            _crf_body,
            grid=(Bc // Bb,),
            in_specs=[
                pl.BlockSpec((L, L, Bb), lambda g: (0, 0, g)),
                pl.BlockSpec((L, L, Bb), lambda g: (0, 0, g)),
                pl.BlockSpec((L, Bb), lambda g: (0, g)),
            ],
            out_specs=pl.BlockSpec((1, 3), lambda g: (0, 0)),
            out_shape=jax.ShapeDtypeStruct((1, 3), jnp.float32),
            scratch_shapes=[
                pltpu.VMEM((L, L, Bb), jnp.float32),
                pltpu.VMEM((L, L, Bb), jnp.float32),
                pltpu.VMEM((L, L, Bb), jnp.float32),
                pltpu.SMEM((1, 1), jnp.float32),
                pltpu.SMEM((1, 1), jnp.float32),
                pltpu.SMEM((1, 1), jnp.float32),
            ],
            interpret=interpret,
        )(stT, mt8T, mr0t)
        accs.append(acc)

    tot = accs[0] if C == 1 else accs[0] + accs[1]
    loss = (tot[0, 0] - tot[0, 1]) / tot[0, 2]
    return loss, scores


def kernel(scores, mask, target):
    return _crf_pallas(scores, mask, target)


# speedup vs baseline: 1.1193x; 1.0179x over previous
import jax
import jax.numpy as jnp
from jax import lax
from jax.experimental import pallas as pl
from jax.experimental.pallas import tpu as pltpu


def _crf_body(stT_ref, mt_ref, mr_ref, loss_ref,
              d_ref, rrev_ref, sk_ref, alogz, agold, alens):
    L = stT_ref.shape[0]
    Bb = stT_ref.shape[2]
    g = pl.program_id(0)
    G = pl.num_programs(0)

    @pl.when(g == 0)
    def _init():
        alogz[0, 0] = 0.0
        agold[0, 0] = 0.0
        alens[0, 0] = 0.0

    x = stT_ref[...]
    gold_part = jnp.sum(jnp.where(mt_ref[...] == 3, x, 0.0))

    lens = jnp.sum(mr_ref[...], axis=0)
    lsum_part = jnp.sum(lens).astype(jnp.float32)
    maxl = jnp.minimum(jnp.max(lens), L - 1)

    isub = lax.broadcasted_iota(jnp.int32, (1, L, 1), 1)
    bit = 1
    while bit < L:
        rolled = jnp.roll(x, -bit, axis=0)
        x = jnp.where((isub & bit) != 0, rolled, x)
        bit *= 2
    sk_ref[...] = x * jnp.float32(1.4426950408889634)

    neg = jnp.full((1, L, Bb), -jnp.inf, dtype=jnp.float32)
    d_ref[0:1, :, :] = neg
    v1 = sk_ref[1:2, 0:L - 1, :]
    d_ref[1:2, 0:L - 1, :] = v1
    rrev_ref[L - 2:L - 1, 1:L, :] = v1

    for w in range(2, L):
        @pl.when(w <= maxl)
        def _step(w=w):
            n = L - w
            left = d_ref[1:w, 0:n, :]
            right = rrev_ref[L - w:L - 1, w:L, :]
            t = left + right
            mx = jnp.max(t, axis=0)
            lse = mx + jnp.log2(jnp.sum(jnp.exp2(t - mx[None, :, :]), axis=0))
            val = lse + sk_ref[w, 0:n, :]
            d_ref[w, 0:n, :] = val
            rrev_ref[L - 1 - w, w:L, :] = val

    dcol = d_ref[:, 0, :]
    lensc = jnp.minimum(lens, L - 1)
    wiota = lax.broadcasted_iota(jnp.int32, (L, Bb), 0)
    contrib = jnp.where(wiota == lensc[None, :], dcol, 0.0)
    logz_part = jnp.sum(contrib) * jnp.float32(0.6931471805599453)

    alogz[0, 0] = alogz[0, 0] + logz_part
    agold[0, 0] = agold[0, 0] + gold_part
    alens[0, 0] = alens[0, 0] + lsum_part

    @pl.when(g == G - 1)
    def _fin():
        acc = jnp.stack([alogz[0, 0], agold[0, 0], alens[0, 0]])
        loss_ref[...] = acc.reshape(1, 3)


def _crf_pallas(scores, mask, target, interpret=False):
    B, L = scores.shape[0], scores.shape[1]
    Bb = min(128, B)
    C = 1
    Bc = B // C

    accs = []
    for c in range(C):
        sl = slice(c * Bc, (c + 1) * Bc)
        stT = jnp.transpose(scores[sl], (2, 1, 0))
        mt8 = (mask[sl].astype(jnp.uint8)
               | (target[sl].astype(jnp.uint8) << 1))
        mt8T = jnp.transpose(mt8, (2, 1, 0))
        mr0t = jnp.transpose(mask[sl, 0, :].astype(jnp.int32), (1, 0))

        acc = pl.pallas_call(
            _crf_body,
            grid=(Bc // Bb,),
            in_specs=[
                pl.BlockSpec((L, L, Bb), lambda g: (0, 0, g)),
                pl.BlockSpec((L, L, Bb), lambda g: (0, 0, g)),
                pl.BlockSpec((L, Bb), lambda g: (0, g)),
            ],
            out_specs=pl.BlockSpec((1, 3), lambda g: (0, 0)),
            out_shape=jax.ShapeDtypeStruct((1, 3), jnp.float32),
            scratch_shapes=[
                pltpu.VMEM((L, L, Bb), jnp.float32),
                pltpu.VMEM((L, L, Bb), jnp.float32),
                pltpu.VMEM((L, L, Bb), jnp.float32),
                pltpu.SMEM((1, 1), jnp.float32),
                pltpu.SMEM((1, 1), jnp.float32),
                pltpu.SMEM((1, 1), jnp.float32),
            ],
            interpret=interpret,
        )(stT, mt8T, mr0t)
        accs.append(acc)

    tot = accs[0] if C == 1 else accs[0] + accs[1]
    loss = (tot[0, 0] - tot[0, 1]) / tot[0, 2]
    return loss, scores


def kernel(scores, mask, target):
    return _crf_pallas(scores, mask, target)


# speedup vs baseline: 1.1380x; 1.0167x over previous
import jax
import jax.numpy as jnp
from jax import lax
from jax.experimental import pallas as pl
from jax.experimental.pallas import tpu as pltpu


def _crf_body(stT_ref, mt_ref, mr_ref, loss_ref,
              d_ref, rrev_ref, sk_ref, alogz, agold, alens):
    L = stT_ref.shape[0]
    Bb = stT_ref.shape[2]
    g = pl.program_id(0)
    G = pl.num_programs(0)

    @pl.when(g == 0)
    def _init():
        alogz[0, 0] = 0.0
        agold[0, 0] = 0.0
        alens[0, 0] = 0.0

    x = stT_ref[...]
    gold_part = jnp.sum(jnp.where(mt_ref[...] == 3, x, 0.0))

    lens = jnp.sum(mr_ref[...], axis=0)
    lsum_part = jnp.sum(lens).astype(jnp.float32)
    maxl = jnp.minimum(jnp.max(lens), L - 1)

    isub = lax.broadcasted_iota(jnp.int32, (1, L, 1), 1)
    bit = 1
    while bit < L:
        rolled = jnp.roll(x, -bit, axis=0)
        x = jnp.where((isub & bit) != 0, rolled, x)
        bit *= 2
    sk_ref[...] = x

    neg = jnp.full((1, L, Bb), -jnp.inf, dtype=jnp.float32)
    d_ref[0:1, :, :] = neg
    v1 = sk_ref[1:2, 0:L - 1, :]
    d_ref[1:2, 0:L - 1, :] = v1
    rrev_ref[L - 2:L - 1, 1:L, :] = v1

    for w in range(2, L):
        @pl.when(w <= maxl)
        def _step(w=w):
            n = L - w
            left = d_ref[1:w, 0:n, :]
            right = rrev_ref[L - w:L - 1, w:L, :]
            t = left + right
            mx = jnp.max(t, axis=0)
            lse = mx + jnp.log(jnp.sum(jnp.exp(t - mx[None, :, :]), axis=0))
            val = lse + sk_ref[w, 0:n, :]
            d_ref[w, 0:n, :] = val
            rrev_ref[L - 1 - w, w:L, :] = val

    dcol = d_ref[:, 0, :]
    lensc = jnp.minimum(lens, L - 1)
    wiota = lax.broadcasted_iota(jnp.int32, (L, Bb), 0)
    contrib = jnp.where(wiota == lensc[None, :], dcol, 0.0)
    logz_part = jnp.sum(contrib)

    alogz[0, 0] = alogz[0, 0] + logz_part
    agold[0, 0] = agold[0, 0] + gold_part
    alens[0, 0] = alens[0, 0] + lsum_part

    @pl.when(g == G - 1)
    def _fin():
        loss = (alogz[0, 0] - agold[0, 0]) / alens[0, 0]
        loss_ref[...] = loss.reshape(1, 1)


def _crf_pallas(scores, mask, target, interpret=False):
    B, L = scores.shape[0], scores.shape[1]
    Bb = min(128, B)
    G = B // Bb

    stT = jnp.transpose(scores, (2, 1, 0))
    mt8 = mask.astype(jnp.uint8) | (target.astype(jnp.uint8) << 1)
    mt8T = jnp.transpose(mt8, (2, 1, 0))
    mr0t = jnp.transpose(mask[:, 0, :].astype(jnp.int32), (1, 0))

    loss2d = pl.pallas_call(
        _crf_body,
        grid=(G,),
        in_specs=[
            pl.BlockSpec((L, L, Bb), lambda g: (0, 0, g)),
            pl.BlockSpec((L, L, Bb), lambda g: (0, 0, g)),
            pl.BlockSpec((L, Bb), lambda g: (0, g)),
        ],
        out_specs=pl.BlockSpec((1, 1), lambda g: (0, 0)),
        out_shape=jax.ShapeDtypeStruct((1, 1), jnp.float32),
        scratch_shapes=[
            pltpu.VMEM((L, L, Bb), jnp.float32),
            pltpu.VMEM((L, L, Bb), jnp.float32),
            pltpu.VMEM((L, L, Bb), jnp.float32),
            pltpu.SMEM((1, 1), jnp.float32),
            pltpu.SMEM((1, 1), jnp.float32),
            pltpu.SMEM((1, 1), jnp.float32),
        ],
        interpret=interpret,
    )(stT, mt8T, mr0t)

    return loss2d[0, 0], scores


def kernel(scores, mask, target):
    return _crf_pallas(scores, mask, target)
